# baseline (speedup 1.0000x reference)
import jax
import jax.numpy as jnp
from jax.experimental import pallas as pl
from jax.experimental.pallas import tpu as pltpu

B, D, K = 4096, 2048, 8
BM = 256
NBLK = B // BM


def _encoder(idx_ref, mask_hbm, x_hbm, w_hbm, b_hbm, out_hbm,
             wtile, btile, mtile, xtile, otile,
             w_sem, m_sem, b_sem, x_sem, o_sem):
    idx = idx_ref[0]

    w_cp = pltpu.make_async_copy(w_hbm, wtile, w_sem)
    w_cp.start()
    m_cp = pltpu.make_async_copy(mask_hbm, mtile, m_sem)
    m_cp.start()
    b_cp = pltpu.make_async_copy(b_hbm, btile, b_sem)
    b_cp.start()

    def x_copy(blk, slot):
        return pltpu.make_async_copy(
            x_hbm.at[pl.ds(blk * BM, BM), :], xtile.at[slot], x_sem.at[slot])

    def o_copy(blk, slot):
        return pltpu.make_async_copy(
            otile.at[slot], out_hbm.at[pl.ds(blk * BM, BM), :],
            o_sem.at[slot])

    x_copy(0, 0).start()
    x_copy(1, 1).start()
    w_cp.wait()
    m_cp.wait()
    b_cp.wait()

    onehot = (jax.lax.broadcasted_iota(jnp.int32, (1, K), 1) == idx
              ).astype(jnp.float32)
    wb = wtile[...].astype(jnp.bfloat16)
    bias = btile[...]

    def step(i, carry):
        slot = jax.lax.rem(i, 3)

        @pl.when(i + 2 < NBLK)
        def _():
            x_copy(i + 2, jax.lax.rem(i + 2, 3)).start()

        x_copy(i, slot).wait()
        xb = xtile[slot].astype(jnp.bfloat16)
        acc = jax.lax.dot_general(
            xb, wb, (((1,), (1,)), ((), ())),
            preferred_element_type=jnp.float32)
        acc = acc + bias
        sel = jnp.sum(mtile[pl.ds(i * BM, BM), :] * onehot, axis=1,
                      keepdims=True)
        keep = sel > 0.5

        oslot = jax.lax.rem(i, 2)
        @pl.when(i >= 2)
        def _():
            o_copy(i - 2, oslot).wait()

        otile[oslot] = jnp.where(keep, acc, 0.0)
        o_copy(i, oslot).start()
        return carry

    jax.lax.fori_loop(0, NBLK, step, 0)
    o_copy(NBLK - 2, 0).wait()
    o_copy(NBLK - 1, 1).wait()


def kernel(input_data, selection_mask, W, bvec, modality_idx):
    idx = jnp.atleast_1d(jnp.asarray(modality_idx, dtype=jnp.int32))
    grid_spec = pltpu.PrefetchScalarGridSpec(
        num_scalar_prefetch=1,
        grid=(1,),
        in_specs=[
            pl.BlockSpec(memory_space=pl.ANY),
            pl.BlockSpec(memory_space=pl.ANY),
            pl.BlockSpec(memory_space=pl.ANY),
            pl.BlockSpec(memory_space=pl.ANY),
        ],
        out_specs=pl.BlockSpec(memory_space=pl.ANY),
        scratch_shapes=[
            pltpu.VMEM((D, D), jnp.float32),
            pltpu.VMEM((1, D), jnp.float32),
            pltpu.VMEM((B, K), jnp.float32),
            pltpu.VMEM((3, BM, D), jnp.float32),
            pltpu.VMEM((2, BM, D), jnp.float32),
            pltpu.SemaphoreType.DMA,
            pltpu.SemaphoreType.DMA,
            pltpu.SemaphoreType.DMA,
            pltpu.SemaphoreType.DMA((3,)),
            pltpu.SemaphoreType.DMA((2,)),
        ],
    )
    return pl.pallas_call(
        _encoder,
        grid_spec=grid_spec,
        out_shape=jax.ShapeDtypeStruct((B, D), jnp.float32),
    )(idx, selection_mask, input_data, W, bvec.reshape(1, D))


# speedup vs baseline: 1.0177x; 1.0177x over previous
import jax
import jax.numpy as jnp
from jax.experimental import pallas as pl
from jax.experimental.pallas import tpu as pltpu

B, D, K = 4096, 2048, 8
BM = 256
NBLK = B // BM


def _encoder(idx_ref, mask_hbm, x_hbm, w_hbm, b_hbm, out_hbm,
             wtile, wbtile, btile, mtile, xtile, otile,
             w_sem, m_sem, b_sem, x_sem, o_sem):
    idx = idx_ref[0]

    w_cp = pltpu.make_async_copy(w_hbm, wtile, w_sem)
    w_cp.start()
    m_cp = pltpu.make_async_copy(mask_hbm, mtile, m_sem)
    m_cp.start()
    b_cp = pltpu.make_async_copy(b_hbm, btile, b_sem)
    b_cp.start()

    def x_copy(blk, slot):
        return pltpu.make_async_copy(
            x_hbm.at[pl.ds(blk * BM, BM), :], xtile.at[slot], x_sem.at[slot])

    def o_copy(blk, slot):
        return pltpu.make_async_copy(
            otile.at[slot], out_hbm.at[pl.ds(blk * BM, BM), :],
            o_sem.at[slot])

    x_copy(0, 0).start()
    x_copy(1, 1).start()
    w_cp.wait()
    m_cp.wait()
    b_cp.wait()

    onehot = (jax.lax.broadcasted_iota(jnp.int32, (1, K), 1) == idx
              ).astype(jnp.float32)
    wbtile[...] = wtile[...].astype(jnp.bfloat16)
    bias = btile[...]

    def step(i, carry):
        slot = jax.lax.rem(i, 3)

        @pl.when(i + 2 < NBLK)
        def _():
            x_copy(i + 2, jax.lax.rem(i + 2, 3)).start()

        x_copy(i, slot).wait()
        xb = xtile[slot].astype(jnp.bfloat16)
        acc = jax.lax.dot_general(
            xb, wbtile[...], (((1,), (1,)), ((), ())),
            preferred_element_type=jnp.float32)
        acc = acc + bias
        sel = jnp.sum(mtile[pl.ds(i * BM, BM), :] * onehot, axis=1,
                      keepdims=True)
        keep = sel > 0.5

        oslot = jax.lax.rem(i, 2)
        @pl.when(i >= 2)
        def _():
            o_copy(i - 2, oslot).wait()

        otile[oslot] = jnp.where(keep, acc, 0.0)
        o_copy(i, oslot).start()
        return carry

    jax.lax.fori_loop(0, NBLK, step, 0)
    o_copy(NBLK - 2, 0).wait()
    o_copy(NBLK - 1, 1).wait()


def kernel(input_data, selection_mask, W, bvec, modality_idx):
    idx = jnp.atleast_1d(jnp.asarray(modality_idx, dtype=jnp.int32))
    grid_spec = pltpu.PrefetchScalarGridSpec(
        num_scalar_prefetch=1,
        grid=(1,),
        in_specs=[
            pl.BlockSpec(memory_space=pl.ANY),
            pl.BlockSpec(memory_space=pl.ANY),
            pl.BlockSpec(memory_space=pl.ANY),
            pl.BlockSpec(memory_space=pl.ANY),
        ],
        out_specs=pl.BlockSpec(memory_space=pl.ANY),
        scratch_shapes=[
            pltpu.VMEM((D, D), jnp.float32),
            pltpu.VMEM((D, D), jnp.bfloat16),
            pltpu.VMEM((1, D), jnp.float32),
            pltpu.VMEM((B, K), jnp.float32),
            pltpu.VMEM((3, BM, D), jnp.float32),
            pltpu.VMEM((2, BM, D), jnp.float32),
            pltpu.SemaphoreType.DMA,
            pltpu.SemaphoreType.DMA,
            pltpu.SemaphoreType.DMA,
            pltpu.SemaphoreType.DMA((3,)),
            pltpu.SemaphoreType.DMA((2,)),
        ],
    )
    return pl.pallas_call(
        _encoder,
        grid_spec=grid_spec,
        out_shape=jax.ShapeDtypeStruct((B, D), jnp.float32),
    )(idx, selection_mask, input_data, W, bvec.reshape(1, D))


# speedup vs baseline: 1.0751x; 1.0564x over previous
import jax
import jax.numpy as jnp
from jax.experimental import pallas as pl
from jax.experimental.pallas import tpu as pltpu

B, D, K = 4096, 2048, 8
BM = 512
NBLK = B // BM


def _encode_block(idx_ref, mask_ref, x_ref, w_hbm, b_ref, out_ref,
                  wtile, wbtile, w_sem):
    i = pl.program_id(0)
    idx = idx_ref[0]

    @pl.when(i == 0)
    def _():
        cp = pltpu.make_async_copy(w_hbm, wtile, w_sem)
        cp.start()
        cp.wait()
        wbtile[...] = wtile[...].astype(jnp.bfloat16)

    onehot = (jax.lax.broadcasted_iota(jnp.int32, (1, K), 1) == idx)
    sel = jnp.sum(mask_ref[...] * onehot.astype(jnp.float32), axis=1,
                  keepdims=True)
    keep = sel > 0.5
    xb = x_ref[...].astype(jnp.bfloat16)
    acc = jax.lax.dot_general(
        xb, wbtile[...], (((1,), (1,)), ((), ())),
        preferred_element_type=jnp.float32)
    acc = acc + b_ref[...]
    out_ref[...] = jnp.where(keep, acc, 0.0)


def kernel(input_data, selection_mask, W, bvec, modality_idx):
    idx = jnp.atleast_1d(jnp.asarray(modality_idx, dtype=jnp.int32))
    grid_spec = pltpu.PrefetchScalarGridSpec(
        num_scalar_prefetch=1,
        grid=(NBLK,),
        in_specs=[
            pl.BlockSpec((BM, K), lambda i, *_: (i, 0)),
            pl.BlockSpec((BM, D), lambda i, *_: (i, 0)),
            pl.BlockSpec(memory_space=pl.ANY),
            pl.BlockSpec((1, D), lambda i, *_: (0, 0)),
        ],
        out_specs=pl.BlockSpec((BM, D), lambda i, *_: (i, 0)),
        scratch_shapes=[
            pltpu.VMEM((D, D), jnp.float32),
            pltpu.VMEM((D, D), jnp.bfloat16),
            pltpu.SemaphoreType.DMA,
        ],
    )
    return pl.pallas_call(
        _encode_block,
        grid_spec=grid_spec,
        out_shape=jax.ShapeDtypeStruct((B, D), jnp.float32),
    )(idx, selection_mask, input_data, W, bvec.reshape(1, D))


# speedup vs baseline: 1.1242x; 1.0457x over previous
import jax
import jax.numpy as jnp
from jax.experimental import pallas as pl
from jax.experimental.pallas import tpu as pltpu

B, D, K = 4096, 2048, 8
BM = 512


def _encode_block(idx_ref, mask_ref, x_ref, w_ref, b_ref, out_ref):
    idx = idx_ref[0]
    onehot = (jax.lax.broadcasted_iota(jnp.int32, (1, K), 1) == idx)
    sel = jnp.sum(mask_ref[...] * onehot.astype(jnp.float32), axis=1,
                  keepdims=True)
    keep = sel > 0.5
    xb = x_ref[...].astype(jnp.bfloat16)
    wb = w_ref[...].astype(jnp.bfloat16)
    acc = jax.lax.dot_general(
        xb, wb, (((1,), (1,)), ((), ())),
        preferred_element_type=jnp.float32)
    acc = acc + b_ref[...]
    out_ref[...] = jnp.where(keep, acc, 0.0)


def kernel(input_data, selection_mask, W, bvec, modality_idx):
    idx = jnp.atleast_1d(jnp.asarray(modality_idx, dtype=jnp.int32))
    grid_spec = pltpu.PrefetchScalarGridSpec(
        num_scalar_prefetch=1,
        grid=(B // BM,),
        in_specs=[
            pl.BlockSpec((BM, K), lambda i, *_: (i, 0)),
            pl.BlockSpec((BM, D), lambda i, *_: (i, 0)),
            pl.BlockSpec((D, D), lambda i, *_: (0, 0)),
            pl.BlockSpec((1, D), lambda i, *_: (0, 0)),
        ],
        out_specs=pl.BlockSpec((BM, D), lambda i, *_: (i, 0)),
    )
    return pl.pallas_call(
        _encode_block,
        grid_spec=grid_spec,
        out_shape=jax.ShapeDtypeStruct((B, D), jnp.float32),
        compiler_params=pltpu.CompilerParams(
            dimension_semantics=("parallel",)),
    )(idx, selection_mask, input_data, W, bvec.reshape(1, D))
